# baseline (speedup 1.0000x reference)
import jax
import jax.numpy as jnp
from jax.experimental import pallas as pl
from jax.experimental.pallas import tpu as pltpu


def _pos_kernel(col_ref, row_ref, o_hbm, scratch, sem):
    nc = col_ref.shape[1]
    w = col_ref.shape[0]
    h = row_ref.shape[0]
    col_t = col_ref[...].T
    row_t = row_ref[...].T
    scratch[:nc] = jnp.broadcast_to(col_t[:, None, :], (nc, h, w))
    scratch[nc:] = jnp.broadcast_to(row_t[:, :, None], (nc, h, w))
    b_total = o_hbm.shape[0]
    for b in range(b_total):
        pltpu.make_async_copy(scratch, o_hbm.at[b], sem).start()
    for b in range(b_total):
        pltpu.make_async_copy(scratch, o_hbm.at[b], sem).wait()


def kernel(x, row_w, col_w):
    b = x.shape[0]
    h, w = x.shape[-2], x.shape[-1]
    nc = row_w.shape[1]
    out = pl.pallas_call(
        _pos_kernel,
        in_specs=[
            pl.BlockSpec(memory_space=pltpu.MemorySpace.VMEM),
            pl.BlockSpec(memory_space=pltpu.MemorySpace.VMEM),
        ],
        out_specs=pl.BlockSpec(memory_space=pl.ANY),
        out_shape=jax.ShapeDtypeStruct((b, 2 * nc, h, w), jnp.float32),
        scratch_shapes=[
            pltpu.VMEM((2 * nc, h, w), jnp.float32),
            pltpu.SemaphoreType.DMA,
        ],
    )(col_w, row_w)
    return out


# speedup vs baseline: 3.1276x; 3.1276x over previous
import jax
import jax.numpy as jnp
from jax.experimental import pallas as pl
from jax.experimental.pallas import tpu as pltpu


def _pos_kernel(col_ref, row_ref, o_hbm, scratch, sem):
    nc = col_ref.shape[1]
    w = col_ref.shape[0]
    h = row_ref.shape[0]
    col_t = col_ref[...].T
    row_t = row_ref[...].T
    scratch[:nc] = jnp.broadcast_to(col_t[:, None, :], (nc, h, w)).reshape(nc, h * w)
    scratch[nc:] = jnp.broadcast_to(row_t[:, :, None], (nc, h, w)).reshape(nc, h * w)
    b_total = o_hbm.shape[0]
    n_sem = sem.shape[0]
    for b in range(b_total):
        pltpu.make_async_copy(scratch, o_hbm.at[b], sem.at[b % n_sem]).start()
    for b in range(b_total):
        pltpu.make_async_copy(scratch, o_hbm.at[b], sem.at[b % n_sem]).wait()


def kernel(x, row_w, col_w):
    b = x.shape[0]
    h, w = x.shape[-2], x.shape[-1]
    nc = row_w.shape[1]
    out = pl.pallas_call(
        _pos_kernel,
        in_specs=[
            pl.BlockSpec(memory_space=pltpu.MemorySpace.VMEM),
            pl.BlockSpec(memory_space=pltpu.MemorySpace.VMEM),
        ],
        out_specs=pl.BlockSpec(memory_space=pl.ANY),
        out_shape=jax.ShapeDtypeStruct((b, 2 * nc, h * w), jnp.float32),
        scratch_shapes=[
            pltpu.VMEM((2 * nc, h * w), jnp.float32),
            pltpu.SemaphoreType.DMA((8,)),
        ],
    )(col_w, row_w)
    return out.reshape(b, 2 * nc, h, w)
